# baseline (device time: 12630 ns/iter reference)
import jax
import jax.numpy as jnp
from jax import lax
from jax.experimental import pallas as pl
from jax.experimental.pallas import tpu as pltpu

N_DEV = 16
M_GLOBAL = 24576
N_CHUNKS = 6


def kernel(x):
    m_per, n = x.shape
    chunk = m_per // N_CHUNKS
    assert chunk * N_CHUNKS == m_per
    x = pltpu.with_memory_space_constraint(x, pltpu.MemorySpace.HBM)

    def body(
        x_hbm,
        out_ref,
        chunk_buf,
        mine_ref,
        comm_ref,
        result_ref,
        copy_sems,
        out_sem,
        send_sems,
        recv_sems,
        credit_sems,
    ):
        my_pos = lax.axis_index("i")

        barrier_sem = pltpu.get_barrier_semaphore()
        for d in range(1, N_DEV):
            tgt = lax.rem(my_pos + d, N_DEV)
            pl.semaphore_signal(
                barrier_sem, inc=1,
                device_id=(tgt,), device_id_type=pl.DeviceIdType.MESH,
            )
            pl.semaphore_signal(
                credit_sems.at[d - 1], inc=1,
                device_id=(tgt,), device_id_type=pl.DeviceIdType.MESH,
            )

        def copy_in(c):
            return pltpu.make_async_copy(
                x_hbm.at[pl.ds(c * chunk, chunk), :],
                chunk_buf.at[c % 2],
                copy_sems.at[c % 2],
            )

        copy_in(0).start()
        copy_in(1).start()
        partial = jnp.zeros((1, n), jnp.float32)
        for c in range(N_CHUNKS):
            copy_in(c).wait()
            partial = partial + jnp.sum(
                chunk_buf[c % 2], axis=0, keepdims=True
            )
            if c + 2 < N_CHUNKS:
                copy_in(c + 2).start()
        mine_ref[...] = partial
        comm_ref[pl.ds(my_pos, 1), :] = partial

        pl.semaphore_wait(barrier_sem, N_DEV - 1)

        rdmas = []
        for d in range(1, N_DEV):
            tgt = lax.rem(my_pos + d, N_DEV)
            pl.semaphore_wait(credit_sems.at[(N_DEV - d) - 1], 1)
            rdma = pltpu.make_async_remote_copy(
                src_ref=mine_ref,
                dst_ref=comm_ref.at[pl.ds(my_pos, 1), :],
                send_sem=send_sems.at[d - 1],
                recv_sem=recv_sems.at[d - 1],
                device_id=(tgt,),
                device_id_type=pl.DeviceIdType.MESH,
            )
            rdma.start()
            rdmas.append(rdma)

        for rdma in rdmas:
            rdma.wait_recv()

        result_ref[...] = jnp.sum(comm_ref[...], axis=0, keepdims=True) * (
            1.0 / M_GLOBAL
        )
        out_cp = pltpu.make_async_copy(result_ref, out_ref, out_sem)
        out_cp.start()

        for rdma in rdmas:
            rdma.wait_send()
        out_cp.wait()

    return pl.pallas_call(
        body,
        out_shape=jax.ShapeDtypeStruct((1, n), jnp.float32),
        in_specs=[pl.BlockSpec(memory_space=pltpu.MemorySpace.HBM)],
        out_specs=pl.BlockSpec(memory_space=pltpu.MemorySpace.HBM),
        scratch_shapes=[
            pltpu.VMEM((2, chunk, n), jnp.float32),
            pltpu.VMEM((1, n), jnp.float32),
            pltpu.VMEM((N_DEV, n), jnp.float32),
            pltpu.VMEM((1, n), jnp.float32),
            pltpu.SemaphoreType.DMA((2,)),
            pltpu.SemaphoreType.DMA,
            pltpu.SemaphoreType.DMA((N_DEV - 1,)),
            pltpu.SemaphoreType.DMA((N_DEV - 1,)),
            pltpu.SemaphoreType.REGULAR((N_DEV - 1,)),
        ],
        compiler_params=pltpu.CompilerParams(collective_id=0),
    )(x)


# device time: 10987 ns/iter; 1.1495x vs baseline; 1.1495x over previous
import jax
import jax.numpy as jnp
from jax import lax
from jax.experimental import pallas as pl
from jax.experimental.pallas import tpu as pltpu

N_DEV = 16
M_GLOBAL = 24576
N_CHUNKS = 6


def kernel(x):
    m_per, n = x.shape
    chunk = m_per // N_CHUNKS
    assert chunk * N_CHUNKS == m_per
    x = pltpu.with_memory_space_constraint(x, pltpu.MemorySpace.HBM)

    def body(
        x_hbm,
        out_ref,
        chunk_buf,
        mine_ref,
        comm_ref,
        result_ref,
        copy_sems,
        out_sem,
        send_sems,
        recv_sems,
        credit_sems,
    ):
        my_pos = lax.axis_index("i")

        barrier_sem = pltpu.get_barrier_semaphore()
        for d in range(1, N_DEV):
            tgt = lax.rem(my_pos + d, N_DEV)
            pl.semaphore_signal(
                barrier_sem, inc=1,
                device_id=(tgt,), device_id_type=pl.DeviceIdType.MESH,
            )
            pl.semaphore_signal(
                credit_sems.at[d - 1], inc=1,
                device_id=(tgt,), device_id_type=pl.DeviceIdType.MESH,
            )

        def copy_in(c):
            return pltpu.make_async_copy(
                x_hbm.at[pl.ds(c * chunk, chunk), :],
                chunk_buf.at[c],
                copy_sems.at[c],
            )

        for c in range(N_CHUNKS):
            copy_in(c).start()
        partial = jnp.zeros((1, n), jnp.float32)
        for c in range(N_CHUNKS):
            copy_in(c).wait()
            partial = partial + jnp.sum(
                chunk_buf[c], axis=0, keepdims=True
            )
        mine_ref[...] = partial
        comm_ref[pl.ds(my_pos, 1), :] = partial

        pl.semaphore_wait(barrier_sem, N_DEV - 1)

        rdmas = []
        for d in range(1, N_DEV):
            tgt = lax.rem(my_pos + d, N_DEV)
            pl.semaphore_wait(credit_sems.at[(N_DEV - d) - 1], 1)
            rdma = pltpu.make_async_remote_copy(
                src_ref=mine_ref,
                dst_ref=comm_ref.at[pl.ds(my_pos, 1), :],
                send_sem=send_sems.at[d - 1],
                recv_sem=recv_sems.at[d - 1],
                device_id=(tgt,),
                device_id_type=pl.DeviceIdType.MESH,
            )
            rdma.start()
            rdmas.append(rdma)

        for rdma in rdmas:
            rdma.wait_recv()

        result_ref[...] = jnp.sum(comm_ref[...], axis=0, keepdims=True) * (
            1.0 / M_GLOBAL
        )
        out_cp = pltpu.make_async_copy(result_ref, out_ref, out_sem)
        out_cp.start()

        for rdma in rdmas:
            rdma.wait_send()
        out_cp.wait()

    return pl.pallas_call(
        body,
        out_shape=jax.ShapeDtypeStruct((1, n), jnp.float32),
        in_specs=[pl.BlockSpec(memory_space=pltpu.MemorySpace.HBM)],
        out_specs=pl.BlockSpec(memory_space=pltpu.MemorySpace.HBM),
        scratch_shapes=[
            pltpu.VMEM((N_CHUNKS, chunk, n), jnp.float32),
            pltpu.VMEM((1, n), jnp.float32),
            pltpu.VMEM((N_DEV, n), jnp.float32),
            pltpu.VMEM((1, n), jnp.float32),
            pltpu.SemaphoreType.DMA((N_CHUNKS,)),
            pltpu.SemaphoreType.DMA,
            pltpu.SemaphoreType.DMA((N_DEV - 1,)),
            pltpu.SemaphoreType.DMA((N_DEV - 1,)),
            pltpu.SemaphoreType.REGULAR((N_DEV - 1,)),
        ],
        compiler_params=pltpu.CompilerParams(collective_id=0),
    )(x)


# device time: 10905 ns/iter; 1.1582x vs baseline; 1.0075x over previous
import jax
import jax.numpy as jnp
from jax import lax
from jax.experimental import pallas as pl
from jax.experimental.pallas import tpu as pltpu

N_DEV = 16
M_GLOBAL = 24576
N_CHUNKS = 6


def kernel(x):
    m_per, n = x.shape
    chunk = m_per // N_CHUNKS
    assert chunk * N_CHUNKS == m_per
    x = pltpu.with_memory_space_constraint(x, pltpu.MemorySpace.HBM)

    def body(
        x_hbm,
        out_ref,
        chunk_buf,
        mine_ref,
        comm_ref,
        result_ref,
        copy_sems,
        out_sem,
        send_sems,
        recv_sems,
        credit_sems,
    ):
        my_pos = lax.axis_index("i")

        for d in range(1, N_DEV):
            tgt = lax.rem(my_pos + d, N_DEV)
            pl.semaphore_signal(
                credit_sems.at[d - 1], inc=1,
                device_id=(tgt,), device_id_type=pl.DeviceIdType.MESH,
            )

        barrier_sem = pltpu.get_barrier_semaphore()
        for nbr_off in (1, N_DEV - 1):
            tgt = lax.rem(my_pos + nbr_off, N_DEV)
            pl.semaphore_signal(
                barrier_sem, inc=1,
                device_id=(tgt,), device_id_type=pl.DeviceIdType.MESH,
            )

        def copy_in(c):
            return pltpu.make_async_copy(
                x_hbm.at[pl.ds(c * chunk, chunk), :],
                chunk_buf.at[c],
                copy_sems.at[c],
            )

        for c in range(N_CHUNKS):
            copy_in(c).start()
        partial = jnp.zeros((1, n), jnp.float32)
        for c in range(N_CHUNKS):
            copy_in(c).wait()
            partial = partial + jnp.sum(
                chunk_buf[c], axis=0, keepdims=True
            )
        mine_ref[...] = partial
        comm_ref[pl.ds(my_pos, 1), :] = partial

        pl.semaphore_wait(barrier_sem, 2)

        rdmas = []
        for d in range(1, N_DEV):
            tgt = lax.rem(my_pos + d, N_DEV)
            pl.semaphore_wait(credit_sems.at[(N_DEV - d) - 1], 1)
            rdma = pltpu.make_async_remote_copy(
                src_ref=mine_ref,
                dst_ref=comm_ref.at[pl.ds(my_pos, 1), :],
                send_sem=send_sems.at[d - 1],
                recv_sem=recv_sems.at[d - 1],
                device_id=(tgt,),
                device_id_type=pl.DeviceIdType.MESH,
            )
            rdma.start()
            rdmas.append(rdma)

        for rdma in rdmas:
            rdma.wait_recv()

        result_ref[...] = jnp.sum(comm_ref[...], axis=0, keepdims=True) * (
            1.0 / M_GLOBAL
        )
        out_cp = pltpu.make_async_copy(result_ref, out_ref, out_sem)
        out_cp.start()

        for rdma in rdmas:
            rdma.wait_send()
        out_cp.wait()

    return pl.pallas_call(
        body,
        out_shape=jax.ShapeDtypeStruct((1, n), jnp.float32),
        in_specs=[pl.BlockSpec(memory_space=pltpu.MemorySpace.HBM)],
        out_specs=pl.BlockSpec(memory_space=pltpu.MemorySpace.HBM),
        scratch_shapes=[
            pltpu.VMEM((N_CHUNKS, chunk, n), jnp.float32),
            pltpu.VMEM((1, n), jnp.float32),
            pltpu.VMEM((N_DEV, n), jnp.float32),
            pltpu.VMEM((1, n), jnp.float32),
            pltpu.SemaphoreType.DMA((N_CHUNKS,)),
            pltpu.SemaphoreType.DMA,
            pltpu.SemaphoreType.DMA((N_DEV - 1,)),
            pltpu.SemaphoreType.DMA((N_DEV - 1,)),
            pltpu.SemaphoreType.REGULAR((N_DEV - 1,)),
        ],
        compiler_params=pltpu.CompilerParams(collective_id=0),
    )(x)
